# Initial kernel scaffold; baseline (speedup 1.0000x reference)
import jax
import jax.numpy as jnp
from jax.experimental import pallas as pl


def kernel(hidden_states, gate_up_proj, down_proj, expert_indices, expert_weights):
    raise NotImplementedError("write your pallas kernel here")


# speedup vs baseline: 1.4349x; 1.4349x over previous
import jax
import jax.numpy as jnp
from jax.experimental import pallas as pl
from jax.experimental.pallas import tpu as pltpu


def _moe_ffn_kernel(idx_ref, wgt_ref, x_ref, gup_ref, down_ref, out_ref):
    e = pl.program_id(0)
    interm = down_ref.shape[1]
    x = x_ref[...]
    gu = jnp.dot(x, gup_ref[0], preferred_element_type=jnp.float32)
    gate = gu[:, :interm]
    up = gu[:, interm:]
    act = gate * jax.nn.sigmoid(gate) * up
    oe = jnp.dot(act, down_ref[0], preferred_element_type=jnp.float32)
    w = jnp.sum(jnp.where(idx_ref[...] == e, wgt_ref[...], 0.0), axis=0)
    contrib = w[:, None] * oe

    @pl.when(e == 0)
    def _init():
        out_ref[...] = contrib

    @pl.when(e != 0)
    def _acc():
        out_ref[...] += contrib


def kernel(hidden_states, gate_up_proj, down_proj, expert_indices, expert_weights):
    num_tokens, hidden = hidden_states.shape
    num_experts, _, two_interm = gate_up_proj.shape
    interm = down_proj.shape[1]
    idx_t = expert_indices.astype(jnp.int32).T
    wgt_t = expert_weights.T
    top_k = idx_t.shape[0]

    return pl.pallas_call(
        _moe_ffn_kernel,
        grid=(num_experts,),
        in_specs=[
            pl.BlockSpec((top_k, num_tokens), lambda e: (0, 0)),
            pl.BlockSpec((top_k, num_tokens), lambda e: (0, 0)),
            pl.BlockSpec((num_tokens, hidden), lambda e: (0, 0)),
            pl.BlockSpec((1, hidden, two_interm), lambda e: (e, 0, 0)),
            pl.BlockSpec((1, interm, hidden), lambda e: (e, 0, 0)),
        ],
        out_specs=pl.BlockSpec((num_tokens, hidden), lambda e: (0, 0)),
        out_shape=jax.ShapeDtypeStruct((num_tokens, hidden), jnp.float32),
    )(idx_t, wgt_t, hidden_states, gate_up_proj, down_proj)
